# baseline (speedup 1.0000x reference)
import functools

import jax
import jax.numpy as jnp
from jax import lax
from jax.experimental import pallas as pl
from jax.experimental.pallas import tpu as pltpu
from jax.experimental.pallas import tpu_sc as plsc

V = 100000
E = 128
H = 256
T = 2048
Bt = 16
NL = 2
N = T * Bt
PADR = NL * Bt
NP = N + 2 * PADR
RB = 4096
EXTN = RB + 2 * PADR

SLICE = N // 2
SEXT = SLICE + 2 * PADR
GS = SLICE // RB
NW = 32
CW = 64
CH = -(-SEXT // CW) // NW + 1
SPAD = NW * CH * CW


def _sc_gather(idx3, table):
    info = plsc.get_sparse_core_info()
    NC = info.num_cores
    mesh = plsc.VectorSubcoreMesh(core_axis_name="c", subcore_axis_name="s")
    NBUF = 6

    @functools.partial(
        pl.kernel,
        mesh=mesh,
        out_type=jax.ShapeDtypeStruct((SPAD, E), jnp.float32),
        scratch_types=(
            [pltpu.VMEM((CH, CW), jnp.int32)]
            + [pltpu.VMEM((CW, E), jnp.float32) for _ in range(NBUF)]
            + [pltpu.SemaphoreType.DMA for _ in range(2 * NBUF)]
        ),
    )
    def k(idx_hbm, table_hbm, out_hbm, idx_v, *bufs_sems):
        bufs = bufs_sems[:NBUF]
        gsem = bufs_sems[NBUF:2 * NBUF]
        ssem = bufs_sems[2 * NBUF:]
        wid = lax.axis_index("s") * NC + lax.axis_index("c")
        base = wid * (CH * CW)
        pltpu.sync_copy(idx_hbm.at[wid], idx_v)
        gcp = [None] * CH
        scp = [None] * CH
        for j in range(min(NBUF - 1, CH)):
            gcp[j] = pltpu.async_copy(
                table_hbm.at[idx_v.at[j]], bufs[j % NBUF], gsem[j % NBUF])
        for j in range(CH):
            p = j % NBUF
            gcp[j].wait()
            scp[j] = pltpu.async_copy(
                bufs[p], out_hbm.at[pl.ds(base + j * CW, CW)], ssem[p])
            nxt = j + NBUF - 1
            if nxt < CH:
                q = nxt % NBUF
                prev = nxt - NBUF
                if prev >= 0:
                    scp[prev].wait()
                gcp[nxt] = pltpu.async_copy(
                    table_hbm.at[idx_v.at[nxt]], bufs[q], gsem[q])
        for j in range(max(0, CH - NBUF), CH):
            scp[j].wait()

    return k(idx3, table)


def _make_body(k):
    bf16 = jnp.bfloat16

    def body(*refs):
        if k == 0:
            (emb_ref, dat_ref, wi_ref, wm_ref, ws_ref,
             flat_ref, acco_ref, accmo_ref,
             h0_ref, h1_ref, h0b_ref, h1b_ref, mh_ref,
             acc_ref, accm_ref) = refs
        else:
            (emb_ref, dat_ref, wi_ref, wm_ref, ws_ref, wt_ref, bt_ref,
             fprev_ref, acci_ref, accmi_ref,
             flat_ref, fs_ref,
             h0_ref, h1_ref, h0b_ref, h1b_ref, mh_ref,
             acc_ref, accm_ref) = refs
        g = pl.program_id(0)
        s = g * RB

        eblk = emb_ref[pl.ds(s, EXTN), :]
        dblk = dat_ref[pl.ds(s, EXTN), :]
        e = jnp.where(dblk != 0, eblk, 0.0).astype(bf16)
        h0 = jnp.tanh(
            jnp.dot(e, wi_ref[...], preferred_element_type=jnp.float32))
        h0_ref[...] = h0
        h0b_ref[...] = h0.astype(bf16)

        def layer(h_ref, hb_ref, n):
            mh_ref[0:n, :] = jnp.dot(hb_ref[0:n, :], wm_ref[...],
                                     preferred_element_type=jnp.float32)
            w = 1.0 + jnp.tanh(
                jnp.sum(h_ref[0: n - Bt, :] * h_ref[Bt:n, :],
                        axis=1, keepdims=True) * (1.0 / 32.0))
            agg = (mh_ref[0: n - 2 * Bt, :] * w[: n - 2 * Bt]
                   + mh_ref[2 * Bt: n, :] * w[Bt: n - Bt])
            return jnp.tanh(
                jnp.dot(hb_ref[Bt: n - Bt, :], ws_ref[...],
                        preferred_element_type=jnp.float32) + agg)

        h1 = layer(h0_ref, h0b_ref, EXTN)
        h1_ref[0: EXTN - 2 * Bt, :] = h1
        h1b_ref[0: EXTN - 2 * Bt, :] = h1.astype(bf16)
        if k == 0:
            @pl.when(g == 0)
            def _():
                h1_ref[0:Bt, :] = jnp.zeros((Bt, H), jnp.float32)
                h1b_ref[0:Bt, :] = jnp.zeros((Bt, H), bf16)
        else:
            @pl.when(g == GS - 1)
            def _():
                h1_ref[EXTN - 3 * Bt: EXTN - 2 * Bt, :] = jnp.zeros(
                    (Bt, H), jnp.float32)
                h1b_ref[EXTN - 3 * Bt: EXTN - 2 * Bt, :] = jnp.zeros(
                    (Bt, H), bf16)

        h2 = layer(h1_ref, h1b_ref, EXTN - 2 * Bt)
        flat_ref[...] = h2.reshape(RB // Bt, Bt, H)

        if k == 0:
            @pl.when(g == 0)
            def _():
                flat_ref[0:1, :, :] = jnp.zeros((1, Bt, H), jnp.float32)
        else:
            @pl.when(g == GS - 1)
            def _():
                flat_ref[RB // Bt - 1: RB // Bt, :, :] = jnp.zeros(
                    (1, Bt, H), jnp.float32)

        maskblk = (dat_ref[pl.ds(s + PADR, RB), :] != 0).astype(jnp.float32)
        hm = h2 * maskblk
        ib = lax.broadcasted_iota(jnp.int32, (Bt, RB), 0)
        ii = lax.broadcasted_iota(jnp.int32, (Bt, RB), 1)
        msel = (ib == (ii & (Bt - 1))).astype(jnp.float32)
        part = jnp.dot(msel, hm, preferred_element_type=jnp.float32)
        partm = jnp.dot(msel, maskblk, preferred_element_type=jnp.float32)

        if k == 0:
            @pl.when(g == 0)
            def _():
                acc_ref[...] = jnp.zeros_like(acc_ref)
                accm_ref[...] = jnp.zeros_like(accm_ref)
        else:
            @pl.when(g == 0)
            def _():
                acc_ref[...] = acci_ref[...]
                accm_ref[...] = accmi_ref[...]

        acc_ref[...] += part
        accm_ref[...] += jnp.broadcast_to(partm, accm_ref.shape)

        if k == 0:
            @pl.when(g == GS - 1)
            def _():
                acco_ref[...] = acc_ref[...]
                accmo_ref[...] = accm_ref[...]
        else:
            @pl.when(g == GS - 1)
            def _():
                denom = jnp.maximum(accm_ref[:, 0:1], 1.0)
                fsm = acc_ref[...] / denom
                z = (jnp.dot(fsm, wt_ref[...],
                             preferred_element_type=jnp.float32)
                     + bt_ref[...])
                fs_ref[...] = jnp.sin(z) / jnp.cos(z)

    return body


_SCRATCH = [
    pltpu.VMEM((EXTN, H), jnp.float32),
    pltpu.VMEM((EXTN, H), jnp.float32),
    pltpu.VMEM((EXTN, H), jnp.bfloat16),
    pltpu.VMEM((EXTN, H), jnp.bfloat16),
    pltpu.VMEM((EXTN, H), jnp.float32),
    pltpu.VMEM((Bt, H), jnp.float32),
    pltpu.VMEM((Bt, 128), jnp.float32),
]

_CONST2 = lambda g: (0, 0)


def _tc_slice0(emb0, dat0, wi, wm, ws, interpret=False):
    return pl.pallas_call(
        _make_body(0),
        grid=(GS,),
        in_specs=[
            pl.BlockSpec((SPAD, E), _CONST2),
            pl.BlockSpec((SEXT, 1), _CONST2),
            pl.BlockSpec((E, H), _CONST2),
            pl.BlockSpec((H, H), _CONST2),
            pl.BlockSpec((H, H), _CONST2),
        ],
        out_specs=[
            pl.BlockSpec((RB // Bt, Bt, H), lambda g: (g, 0, 0)),
            pl.BlockSpec((Bt, H), _CONST2),
            pl.BlockSpec((Bt, 128), _CONST2),
        ],
        out_shape=[
            jax.ShapeDtypeStruct((T, Bt, H), jnp.float32),
            jax.ShapeDtypeStruct((Bt, H), jnp.float32),
            jax.ShapeDtypeStruct((Bt, 128), jnp.float32),
        ],
        scratch_shapes=list(_SCRATCH),
        compiler_params=pltpu.CompilerParams(
            dimension_semantics=("arbitrary",)),
        interpret=interpret,
    )(emb0, dat0, wi, wm, ws)


def _tc_slice1(emb1, dat1, wi, wm, ws, wt, bt, flat_prev, acc_in, accm_in,
               interpret=False):
    return pl.pallas_call(
        _make_body(1),
        grid=(GS,),
        in_specs=[
            pl.BlockSpec((SPAD, E), _CONST2),
            pl.BlockSpec((SEXT, 1), _CONST2),
            pl.BlockSpec((E, H), _CONST2),
            pl.BlockSpec((H, H), _CONST2),
            pl.BlockSpec((H, H), _CONST2),
            pl.BlockSpec((H, NL * H), _CONST2),
            pl.BlockSpec((1, NL * H), _CONST2),
            pl.BlockSpec(memory_space=pl.ANY),
            pl.BlockSpec((Bt, H), _CONST2),
            pl.BlockSpec((Bt, 128), _CONST2),
        ],
        out_specs=[
            pl.BlockSpec((RB // Bt, Bt, H), lambda g: (g + GS, 0, 0)),
            pl.BlockSpec((Bt, NL * H), _CONST2),
        ],
        out_shape=[
            jax.ShapeDtypeStruct((T, Bt, H), jnp.float32),
            jax.ShapeDtypeStruct((Bt, NL * H), jnp.float32),
        ],
        scratch_shapes=list(_SCRATCH),
        input_output_aliases={7: 0},
        compiler_params=pltpu.CompilerParams(
            dimension_semantics=("arbitrary",)),
        interpret=interpret,
    )(emb1, dat1, wi, wm, ws, wt, bt, flat_prev, acc_in, accm_in)


def kernel(data, embed, W_in, W_msg, W_self, W_tr, b_tr):
    data = data.astype(jnp.int32)
    dflat = data.reshape(-1)
    dpad = jnp.pad(dflat, (PADR, PADR))
    zfill = jnp.zeros((SPAD - SEXT,), jnp.int32)
    idx0 = jnp.concatenate([dpad[0:SEXT], zfill]).reshape(NW, CH, CW)
    idx1 = jnp.concatenate([dpad[NP - SEXT:], zfill]).reshape(NW, CH, CW)
    emb0 = _sc_gather(idx0, embed)
    emb1 = _sc_gather(idx1, embed)
    dpad2 = dpad.reshape(NP, 1)
    dat0 = dpad2[0:SEXT]
    dat1 = dpad2[NP - SEXT:]

    wi = W_in.astype(jnp.bfloat16)
    wm = (0.5 * W_msg).astype(jnp.bfloat16)
    ws = W_self.astype(jnp.bfloat16)
    bt2 = b_tr.reshape(1, NL * H)

    flat0, acc, accm = _tc_slice0(emb0, dat0, wi, wm, ws)
    flat, fs = _tc_slice1(emb1, dat1, wi, wm, ws, W_tr, bt2,
                          flat0, acc, accm)
    fs3 = fs.reshape(Bt, NL, H).transpose(1, 0, 2)
    return (flat, fs3, jnp.zeros_like(fs3))


# speedup vs baseline: 2.6310x; 2.6310x over previous
import functools

import jax
import jax.numpy as jnp
from jax import lax
from jax.experimental import pallas as pl
from jax.experimental.pallas import tpu as pltpu
from jax.experimental.pallas import tpu_sc as plsc

V = 100000
E = 128
H = 256
T = 2048
Bt = 16
NL = 2
N = T * Bt
PADR = NL * Bt
NP = N + 2 * PADR
G = 8
RB = N // G
EXTN = RB + 2 * PADR


def _sc_gather(idx3, table):
    info = plsc.get_sparse_core_info()
    NC, NS = info.num_cores, info.num_subcores
    NW = NC * NS
    CH, CW = idx3.shape[1], idx3.shape[2]
    mesh = plsc.VectorSubcoreMesh(core_axis_name="c", subcore_axis_name="s")

    NBUF = 6

    @functools.partial(
        pl.kernel,
        mesh=mesh,
        out_type=jax.ShapeDtypeStruct((NP, E), jnp.float32),
        scratch_types=(
            [pltpu.VMEM((CH, CW), jnp.int32)]
            + [pltpu.VMEM((CW, E), jnp.float32) for _ in range(NBUF)]
            + [pltpu.SemaphoreType.DMA for _ in range(2 * NBUF)]
        ),
    )
    def k(idx_hbm, table_hbm, out_hbm, idx_v, *bufs_sems):
        bufs = bufs_sems[:NBUF]
        gsem = bufs_sems[NBUF:2 * NBUF]
        ssem = bufs_sems[2 * NBUF:]
        wid = lax.axis_index("s") * NC + lax.axis_index("c")
        base = wid * (CH * CW) + PADR
        pltpu.sync_copy(idx_hbm.at[wid], idx_v)
        gcp = [None] * CH
        scp = [None] * CH
        for j in range(min(NBUF - 1, CH)):
            gcp[j] = pltpu.async_copy(
                table_hbm.at[idx_v.at[j]], bufs[j % NBUF], gsem[j % NBUF])
        for j in range(CH):
            p = j % NBUF
            gcp[j].wait()
            scp[j] = pltpu.async_copy(
                bufs[p], out_hbm.at[pl.ds(base + j * CW, CW)], ssem[p])
            nxt = j + NBUF - 1
            if nxt < CH:
                q = nxt % NBUF
                prev = nxt - NBUF
                if prev >= 0:
                    scp[prev].wait()
                gcp[nxt] = pltpu.async_copy(
                    table_hbm.at[idx_v.at[nxt]], bufs[q], gsem[q])
        for j in range(max(0, CH - NBUF), CH):
            scp[j].wait()

    return k(idx3, table)


def _dense_body(emb_ref, dat_ref, wi_ref, wm_ref, ws_ref, wt_ref, bt_ref,
                flat_ref, fs_ref, h0_ref, h1_ref, h0b_ref, h1b_ref, mh_ref,
                acc_ref, accm_ref):
    g = pl.program_id(0)
    s = g * RB
    first = g == 0
    last = g == G - 1
    bf16 = jnp.bfloat16

    eblk = emb_ref[pl.ds(s, EXTN), :]
    dblk = dat_ref[pl.ds(s, EXTN), :]
    e = jnp.where(dblk != 0, eblk, 0.0).astype(bf16)
    h0 = jnp.tanh(jnp.dot(e, wi_ref[...], preferred_element_type=jnp.float32))
    h0_ref[...] = h0
    h0b_ref[...] = h0.astype(bf16)

    def layer(h_ref, hb_ref, n):
        mh_ref[0:n, :] = jnp.dot(hb_ref[0:n, :], wm_ref[...],
                                 preferred_element_type=jnp.float32)
        w = 1.0 + jnp.tanh(
            jnp.sum(h_ref[0: n - Bt, :] * h_ref[Bt:n, :],
                    axis=1, keepdims=True) * (1.0 / 32.0))
        agg = (mh_ref[0: n - 2 * Bt, :] * w[: n - 2 * Bt]
               + mh_ref[2 * Bt: n, :] * w[Bt: n - Bt])
        return jnp.tanh(
            jnp.dot(hb_ref[Bt: n - Bt, :], ws_ref[...],
                    preferred_element_type=jnp.float32) + agg)

    h1 = layer(h0_ref, h0b_ref, EXTN)
    h1_ref[0: EXTN - 2 * Bt, :] = h1
    h1b_ref[0: EXTN - 2 * Bt, :] = h1.astype(bf16)
    @pl.when(first)
    def _():
        h1_ref[0:Bt, :] = jnp.zeros((Bt, H), jnp.float32)
        h1b_ref[0:Bt, :] = jnp.zeros((Bt, H), bf16)

    @pl.when(last)
    def _():
        h1_ref[EXTN - 3 * Bt: EXTN - 2 * Bt, :] = jnp.zeros((Bt, H),
                                                            jnp.float32)
        h1b_ref[EXTN - 3 * Bt: EXTN - 2 * Bt, :] = jnp.zeros((Bt, H), bf16)

    h2 = layer(h1_ref, h1b_ref, EXTN - 2 * Bt)
    flat_ref[...] = h2.reshape(RB // Bt, Bt, H)

    @pl.when(first)
    def _():
        flat_ref[0:1, :, :] = jnp.zeros((1, Bt, H), jnp.float32)

    @pl.when(last)
    def _():
        flat_ref[RB // Bt - 1: RB // Bt, :, :] = jnp.zeros((1, Bt, H),
                                                           jnp.float32)

    maskblk = (dat_ref[pl.ds(s + PADR, RB), :] != 0).astype(jnp.float32)
    hm = h2 * maskblk
    ib = lax.broadcasted_iota(jnp.int32, (Bt, RB), 0)
    ii = lax.broadcasted_iota(jnp.int32, (Bt, RB), 1)
    msel = (ib == (ii & (Bt - 1))).astype(jnp.float32)
    part = jnp.dot(msel, hm, preferred_element_type=jnp.float32)
    partm = jnp.dot(msel, maskblk, preferred_element_type=jnp.float32)

    @pl.when(g == 0)
    def _():
        acc_ref[...] = jnp.zeros_like(acc_ref)
        accm_ref[...] = jnp.zeros_like(accm_ref)

    acc_ref[...] += part
    accm_ref[...] += jnp.broadcast_to(partm, accm_ref.shape)

    @pl.when(g == G - 1)
    def _():
        denom = jnp.maximum(accm_ref[:, 0:1], 1.0)
        fsm = acc_ref[...] / denom
        z = (jnp.dot(fsm, wt_ref[...], preferred_element_type=jnp.float32)
             + bt_ref[...])
        fs_ref[...] = jnp.sin(z) / jnp.cos(z)


def _tc_dense(emb_pad, data_pad, W_in, W_msg, W_self, W_tr, b_tr2,
              interpret=False):
    flat2d, fs = pl.pallas_call(
        _dense_body,
        grid=(G,),
        in_specs=[
            pl.BlockSpec((NP, E), lambda g: (0, 0)),
            pl.BlockSpec((NP, 1), lambda g: (0, 0)),
            pl.BlockSpec((E, H), lambda g: (0, 0)),
            pl.BlockSpec((H, H), lambda g: (0, 0)),
            pl.BlockSpec((H, H), lambda g: (0, 0)),
            pl.BlockSpec((H, NL * H), lambda g: (0, 0)),
            pl.BlockSpec((1, NL * H), lambda g: (0, 0)),
        ],
        out_specs=[
            pl.BlockSpec((RB // Bt, Bt, H), lambda g: (g, 0, 0)),
            pl.BlockSpec((Bt, NL * H), lambda g: (0, 0)),
        ],
        out_shape=[
            jax.ShapeDtypeStruct((T, Bt, H), jnp.float32),
            jax.ShapeDtypeStruct((Bt, NL * H), jnp.float32),
        ],
        scratch_shapes=[
            pltpu.VMEM((EXTN, H), jnp.float32),
            pltpu.VMEM((EXTN, H), jnp.float32),
            pltpu.VMEM((EXTN, H), jnp.bfloat16),
            pltpu.VMEM((EXTN, H), jnp.bfloat16),
            pltpu.VMEM((EXTN, H), jnp.float32),
            pltpu.VMEM((Bt, H), jnp.float32),
            pltpu.VMEM((Bt, 128), jnp.float32),
        ],
        compiler_params=pltpu.CompilerParams(
            dimension_semantics=("arbitrary",)),
        interpret=interpret,
    )(emb_pad, data_pad, W_in, W_msg, W_self, W_tr, b_tr2)
    return flat2d, fs


def kernel(data, embed, W_in, W_msg, W_self, W_tr, b_tr):
    data = data.astype(jnp.int32)
    idx3 = data.reshape(-1).reshape(32, N // 32 // 128, 128)
    emb_pad = _sc_gather(idx3, embed)
    data_pad = jnp.pad(data.reshape(-1), (PADR, PADR)).reshape(NP, 1)
    flat, fs = _tc_dense(emb_pad, data_pad,
                         W_in.astype(jnp.bfloat16),
                         (0.5 * W_msg).astype(jnp.bfloat16),
                         W_self.astype(jnp.bfloat16),
                         W_tr, b_tr.reshape(1, NL * H))
    fs3 = fs.reshape(Bt, NL, H).transpose(1, 0, 2)
    return (flat, fs3, jnp.zeros_like(fs3))
